# baseline (speedup 1.0000x reference)
import jax
import jax.numpy as jnp
from jax import lax
from jax.experimental import pallas as pl
from jax.experimental.pallas import tpu as pltpu
from jax.experimental.pallas import tpu_sc as plsc

STATE_LEN = 1024
SEQ_LEN = 2048
NPOI = 4096

NUM_CORES = 2
NUM_SUBCORES = 16
LANES = 16
NW = NUM_CORES * NUM_SUBCORES
ROWS_PER_W = STATE_LEN // NW
CHUNK = 8
NCHUNK = ROWS_PER_W // CHUNK
U1 = 4
J1 = SEQ_LEN // (LANES * U1)

TC_BLOCK_ROWS = 256


NRING = 8
RSTEPS = STATE_LEN // NRING


def _rowgather_body(cur_ref, d_any, r_any, *sems):
    def copy_desc(i, k):
        return pltpu.make_async_copy(
            d_any.at[pl.ds(cur_ref[i], 1)], r_any.at[pl.ds(i, 1)], sems[k]
        )

    def body(j, carry):
        for k in range(NRING):
            i = j * NRING + k

            @pl.when(j > 0)
            def _(j=j, k=k, i=i):
                copy_desc(i - NRING, k).wait()

            copy_desc(i, k).start()
        return carry

    lax.fori_loop(0, RSTEPS, body, 0)
    for k in range(NRING):
        copy_desc((RSTEPS - 1) * NRING + k, k).wait()


def _colgather_body(his_hbm, r_hbm, g_hbm,
                    his_v, rows_a, rows_b, g_v, sem_a, sem_b):
    wid = lax.axis_index("s") * NUM_CORES + lax.axis_index("c")
    base = wid * ROWS_PER_W

    pltpu.sync_copy(his_hbm, his_v)

    bufs = (rows_a, rows_b)
    sems = (sem_a, sem_b)

    def start_stage(c):
        return pltpu.async_copy(
            r_hbm.at[pl.ds(base + c * CHUNK, CHUNK)],
            bufs[c % 2], sems[c % 2],
        )

    row_ids = [jnp.full((LANES,), r, jnp.int32) for r in range(CHUNK)]

    pending = start_stage(0)
    for c in range(NCHUNK):
        pending.wait()
        if c + 1 < NCHUNK:
            pending = start_stage(c + 1)
        rows_v = bufs[c % 2]

        def colgather(j, carry, rows_v=rows_v):
            for u in range(U1):
                off = (j * U1 + u) * LANES
                idx = his_v[pl.ds(off, LANES)]
                for r in range(CHUNK):
                    g_v[r, pl.ds(off, LANES)] = plsc.load_gather(
                        rows_v, [row_ids[r], idx]
                    )
            return carry

        lax.fori_loop(0, J1, colgather, 0)
        pltpu.sync_copy(g_v, g_hbm.at[pl.ds(base + c * CHUNK, CHUNK)])


def _softmax_body(g_ref, o_ref):
    d = g_ref[...]
    nz = d != 0.0
    e = jnp.where(nz, 1.0 / jnp.where(nz, d, 1.0), 1e-6)
    m = jnp.max(e, axis=-1, keepdims=True)
    p = jnp.exp(e - m)
    o_ref[...] = p / jnp.sum(p, axis=-1, keepdims=True)


@jax.jit
def kernel(his, cur, poi_distance_mat):
    rowgather = pl.pallas_call(
        _rowgather_body,
        grid_spec=pltpu.PrefetchScalarGridSpec(
            num_scalar_prefetch=1,
            grid=(1,),
            in_specs=[pl.BlockSpec(memory_space=pl.ANY)],
            out_specs=pl.BlockSpec(memory_space=pl.ANY),
            scratch_shapes=[pltpu.SemaphoreType.DMA] * NRING,
        ),
        out_shape=jax.ShapeDtypeStruct((STATE_LEN, NPOI), jnp.float32),
    )
    r = rowgather(cur.astype(jnp.int32), poi_distance_mat)

    colgather = pl.kernel(
        _colgather_body,
        out_type=jax.ShapeDtypeStruct((STATE_LEN, SEQ_LEN), jnp.float32),
        mesh=plsc.VectorSubcoreMesh(core_axis_name="c", subcore_axis_name="s"),
        scratch_types=[
            pltpu.VMEM((SEQ_LEN,), jnp.int32),
            pltpu.VMEM((CHUNK, NPOI), jnp.float32),
            pltpu.VMEM((CHUNK, NPOI), jnp.float32),
            pltpu.VMEM((CHUNK, SEQ_LEN), jnp.float32),
            pltpu.SemaphoreType.DMA,
            pltpu.SemaphoreType.DMA,
        ],
        compiler_params=pltpu.CompilerParams(
            use_tc_tiling_on_sc=False, needs_layout_passes=False
        ),
    )
    g = colgather(his.astype(jnp.int32), r)

    softmax = pl.pallas_call(
        _softmax_body,
        grid=(STATE_LEN // TC_BLOCK_ROWS,),
        in_specs=[
            pl.BlockSpec((TC_BLOCK_ROWS, SEQ_LEN), lambda i: (i, 0)),
        ],
        out_specs=pl.BlockSpec((TC_BLOCK_ROWS, SEQ_LEN), lambda i: (i, 0)),
        out_shape=jax.ShapeDtypeStruct((STATE_LEN, SEQ_LEN), jnp.float32),
    )
    return softmax(g)


# speedup vs baseline: 5.0981x; 5.0981x over previous
import jax
import jax.numpy as jnp
from jax import lax
from jax.experimental import pallas as pl
from jax.experimental.pallas import tpu as pltpu
from jax.experimental.pallas import tpu_sc as plsc

STATE_LEN = 1024
SEQ_LEN = 2048
NPOI = 4096

NUM_CORES = 2
NUM_SUBCORES = 16
LANES = 16
NW = NUM_CORES * NUM_SUBCORES
ROWS_PER_W = STATE_LEN // NW
CHUNK = 8
NCHUNK = ROWS_PER_W // CHUNK
U1 = 8
J1 = SEQ_LEN // (LANES * U1)

TC_BLOCK_ROWS = 256


def _gather_body(his_hbm, cur_hbm, d_hbm, g_hbm,
                 his_v, cur_v, rows_a, rows_b, g_a, g_b,
                 sem_a, sem_b, sem_oa, sem_ob):
    wid = lax.axis_index("s") * NUM_CORES + lax.axis_index("c")
    base = wid * ROWS_PER_W

    pltpu.sync_copy(his_hbm, his_v)
    pltpu.sync_copy(cur_hbm.at[pl.ds(base, ROWS_PER_W)], cur_v)

    bufs = (rows_a, rows_b)
    sems = (sem_a, sem_b)

    def start_gather(c):
        return pltpu.async_copy(
            d_hbm.at[cur_v.at[pl.ds(c * CHUNK, CHUNK)]],
            bufs[c % 2], sems[c % 2],
        )

    row_ids = [jnp.full((LANES,), r, jnp.int32) for r in range(CHUNK)]
    gbufs = (g_a, g_b)
    osems = (sem_oa, sem_ob)

    pending = start_gather(0)
    out_pending = [None, None]
    for c in range(NCHUNK):
        pending.wait()
        if c + 1 < NCHUNK:
            pending = start_gather(c + 1)
        rows_v = bufs[c % 2]
        g_v = gbufs[c % 2]
        if out_pending[c % 2] is not None:
            out_pending[c % 2].wait()

        def colgather(j, carry, rows_v=rows_v, g_v=g_v):
            for u in range(U1):
                off = (j * U1 + u) * LANES
                idx = his_v[pl.ds(off, LANES)]
                for r in range(CHUNK):
                    g_v[r, pl.ds(off, LANES)] = plsc.load_gather(
                        rows_v, [row_ids[r], idx]
                    )
            return carry

        lax.fori_loop(0, J1, colgather, 0)
        out_pending[c % 2] = pltpu.async_copy(
            g_v, g_hbm.at[pl.ds(base + c * CHUNK, CHUNK)], osems[c % 2]
        )
    for d in out_pending:
        if d is not None:
            d.wait()


def _softmax_body(g_ref, o_ref):
    d = g_ref[...]
    nz = d != 0.0
    e = jnp.where(nz, 1.0 / jnp.where(nz, d, 1.0), 1e-6)
    m = jnp.max(e, axis=-1, keepdims=True)
    p = jnp.exp(e - m)
    o_ref[...] = p / jnp.sum(p, axis=-1, keepdims=True)


@jax.jit
def kernel(his, cur, poi_distance_mat):
    gather = pl.kernel(
        _gather_body,
        out_type=jax.ShapeDtypeStruct((STATE_LEN, SEQ_LEN), jnp.float32),
        mesh=plsc.VectorSubcoreMesh(core_axis_name="c", subcore_axis_name="s"),
        scratch_types=[
            pltpu.VMEM((SEQ_LEN,), jnp.int32),
            pltpu.VMEM((ROWS_PER_W,), jnp.int32),
            pltpu.VMEM((CHUNK, NPOI), jnp.float32),
            pltpu.VMEM((CHUNK, NPOI), jnp.float32),
            pltpu.VMEM((CHUNK, SEQ_LEN), jnp.float32),
            pltpu.VMEM((CHUNK, SEQ_LEN), jnp.float32),
            pltpu.SemaphoreType.DMA,
            pltpu.SemaphoreType.DMA,
            pltpu.SemaphoreType.DMA,
            pltpu.SemaphoreType.DMA,
        ],
        compiler_params=pltpu.CompilerParams(
            use_tc_tiling_on_sc=False, needs_layout_passes=False
        ),
    )
    g = gather(his.astype(jnp.int32), cur.astype(jnp.int32), poi_distance_mat)

    softmax = pl.pallas_call(
        _softmax_body,
        grid=(STATE_LEN // TC_BLOCK_ROWS,),
        in_specs=[
            pl.BlockSpec((TC_BLOCK_ROWS, SEQ_LEN), lambda i: (i, 0)),
        ],
        out_specs=pl.BlockSpec((TC_BLOCK_ROWS, SEQ_LEN), lambda i: (i, 0)),
        out_shape=jax.ShapeDtypeStruct((STATE_LEN, SEQ_LEN), jnp.float32),
    )
    return softmax(g)


# speedup vs baseline: 5.4671x; 1.0724x over previous
import jax
import jax.numpy as jnp
from jax import lax
from jax.experimental import pallas as pl
from jax.experimental.pallas import tpu as pltpu
from jax.experimental.pallas import tpu_sc as plsc

STATE_LEN = 1024
SEQ_LEN = 2048
NPOI = 4096

NUM_CORES = 2
NUM_SUBCORES = 16
LANES = 16
NW = NUM_CORES * NUM_SUBCORES
ROWS_PER_W = STATE_LEN // NW
CHUNK = 8
NCHUNK = ROWS_PER_W // CHUNK
U1 = 8
J1 = SEQ_LEN // (LANES * U1)

TC_BLOCK_ROWS = 256


def _gather_body(his_hbm, cur_hbm, d_hbm, g_hbm,
                 his_v, cur_v, rows_a, rows_b, g_a, g_b,
                 sem_a, sem_b, sem_oa, sem_ob):
    wid = lax.axis_index("s") * NUM_CORES + lax.axis_index("c")
    base = wid * ROWS_PER_W

    pltpu.sync_copy(his_hbm, his_v)
    pltpu.sync_copy(cur_hbm.at[pl.ds(base, ROWS_PER_W)], cur_v)

    bufs = (rows_a, rows_b)
    sems = (sem_a, sem_b)

    def start_gather(c):
        return pltpu.async_copy(
            d_hbm.at[cur_v.at[pl.ds(c * CHUNK, CHUNK)]],
            bufs[c % 2], sems[c % 2],
        )

    row_ids = [jnp.full((LANES,), r, jnp.int32) for r in range(CHUNK)]
    gbufs = (g_a, g_b)
    osems = (sem_oa, sem_ob)

    pending = start_gather(0)
    out_pending = [None, None]
    for c in range(NCHUNK):
        pending.wait()
        if c + 1 < NCHUNK:
            pending = start_gather(c + 1)
        rows_v = bufs[c % 2]
        g_v = gbufs[c % 2]
        if out_pending[c % 2] is not None:
            out_pending[c % 2].wait()

        def colgather(j, carry, rows_v=rows_v, g_v=g_v):
            for u in range(U1):
                off = j * (U1 * LANES) + u * LANES
                idx = his_v[pl.ds(off, LANES)]
                for r in range(CHUNK):
                    g_v[r, j, u * LANES:(u + 1) * LANES] = plsc.load_gather(
                        rows_v, [row_ids[r], idx]
                    )
            return carry

        lax.fori_loop(0, J1, colgather, 0)
        out_pending[c % 2] = pltpu.async_copy(
            g_v, g_hbm.at[pl.ds(base + c * CHUNK, CHUNK)], osems[c % 2]
        )
    for d in out_pending:
        if d is not None:
            d.wait()


def _softmax_body(g_ref, o_ref):
    d = g_ref[...]
    nz = d != 0.0
    e = jnp.where(nz, 1.0 / jnp.where(nz, d, 1.0), 1e-6)
    m = jnp.max(e, axis=(1, 2), keepdims=True)
    p = jnp.exp(e - m)
    s = jnp.sum(p, axis=(1, 2), keepdims=True)
    o_ref[...] = (p / s).reshape(TC_BLOCK_ROWS, SEQ_LEN)


@jax.jit
def kernel(his, cur, poi_distance_mat):
    gather = pl.kernel(
        _gather_body,
        out_type=jax.ShapeDtypeStruct((STATE_LEN, SEQ_LEN // 128, 128), jnp.float32),
        mesh=plsc.VectorSubcoreMesh(core_axis_name="c", subcore_axis_name="s"),
        scratch_types=[
            pltpu.VMEM((SEQ_LEN,), jnp.int32),
            pltpu.VMEM((ROWS_PER_W,), jnp.int32),
            pltpu.VMEM((CHUNK, NPOI), jnp.float32),
            pltpu.VMEM((CHUNK, NPOI), jnp.float32),
            pltpu.VMEM((CHUNK, SEQ_LEN // 128, 128), jnp.float32),
            pltpu.VMEM((CHUNK, SEQ_LEN // 128, 128), jnp.float32),
            pltpu.SemaphoreType.DMA,
            pltpu.SemaphoreType.DMA,
            pltpu.SemaphoreType.DMA,
            pltpu.SemaphoreType.DMA,
        ],
        compiler_params=pltpu.CompilerParams(
            use_tc_tiling_on_sc=False, needs_layout_passes=False
        ),
    )
    g = gather(his.astype(jnp.int32), cur.astype(jnp.int32), poi_distance_mat)

    softmax = pl.pallas_call(
        _softmax_body,
        grid=(STATE_LEN // TC_BLOCK_ROWS,),
        in_specs=[
            pl.BlockSpec((TC_BLOCK_ROWS, SEQ_LEN // 128, 128),
                         lambda i: (i, 0, 0)),
        ],
        out_specs=pl.BlockSpec((TC_BLOCK_ROWS, SEQ_LEN), lambda i: (i, 0)),
        out_shape=jax.ShapeDtypeStruct((STATE_LEN, SEQ_LEN), jnp.float32),
    )
    return softmax(g)
